# Initial kernel scaffold; baseline (speedup 1.0000x reference)
import jax
import jax.numpy as jnp
from jax.experimental import pallas as pl


def kernel(features, rois):
    raise NotImplementedError("write your pallas kernel here")


# speedup vs baseline: 29.0896x; 29.0896x over previous
import functools

import jax
import jax.numpy as jnp
from jax.experimental import pallas as pl
from jax.experimental.pallas import tpu as pltpu

_OH, _OW = 7, 7
_SCALE = 0.0625

_INTERPRET = False


def _tc_body(b_ref, y1_ref, h_ref, x1_ref, w_ref, feat_ref, out_ref, scratch,
             *, H, W, C):
    i = pl.program_id(0)
    b = b_ref[i]
    y1 = y1_ref[i]
    h = h_ref[i]
    x1 = x1_ref[i]
    w = w_ref[i]

    for j in range(_OH):
        rs = y1 + (j * h) // _OH
        re = y1 + ((j + 1) * h + _OH - 1) // _OH
        rs_c = jnp.minimum(rs, H - 8)
        slab = feat_ref[b, pl.ds(rs_c, 8), :, :]
        row = jax.lax.broadcasted_iota(jnp.int32, (8, W, C), 0) + rs_c
        masked = jnp.where((row >= rs) & (row < re), slab, -jnp.inf)
        scratch[j, :, :] = jnp.max(masked, axis=0)

    for k in range(_OW):
        cs = x1 + (k * w) // _OW
        ce = x1 + ((k + 1) * w + _OW - 1) // _OW
        cs_a = pl.multiple_of(jnp.minimum(8 * (cs // 8), 48), 8)
        slab2 = scratch[:, pl.ds(cs_a, 28), :]
        col = jax.lax.broadcasted_iota(jnp.int32, (_OH, 28, C), 1) + cs_a
        masked2 = jnp.where((col >= cs) & (col < ce), slab2, -jnp.inf)
        out_ref[0, :, k, :] = jnp.max(masked2, axis=1)


def kernel(features, rois):
    B, C, H, W = features.shape
    N = rois.shape[0]

    feats = jnp.transpose(features, (0, 2, 3, 1))

    bidx = rois[:, 0].astype(jnp.int32)
    coords = (rois[:, 1:5] * _SCALE).astype(jnp.int32)
    x1 = jnp.clip(coords[:, 0], 0, W - 1)
    y1 = jnp.clip(coords[:, 1], 0, H - 1)
    x2 = jnp.clip(coords[:, 2], 0, W - 1)
    y2 = jnp.clip(coords[:, 3], 0, H - 1)
    h = y2 - y1 + 1
    w = x2 - x1 + 1

    grid_spec = pltpu.PrefetchScalarGridSpec(
        num_scalar_prefetch=5,
        grid=(N,),
        in_specs=[
            pl.BlockSpec((B, H, W, C), lambda i, *refs: (0, 0, 0, 0)),
        ],
        out_specs=pl.BlockSpec((1, _OH, _OW, C),
                               lambda i, *refs: (i, 0, 0, 0)),
        scratch_shapes=[pltpu.VMEM((_OH, W, C), jnp.float32)],
    )

    out = pl.pallas_call(
        functools.partial(_tc_body, H=H, W=W, C=C),
        grid_spec=grid_spec,
        out_shape=jax.ShapeDtypeStruct((N, _OH, _OW, C), jnp.float32),
        interpret=_INTERPRET,
    )(bidx, y1, h, x1, w, feats)

    return jnp.transpose(out, (0, 3, 1, 2))
